# Initial kernel scaffold; baseline (speedup 1.0000x reference)
import jax
import jax.numpy as jnp
from jax.experimental import pallas as pl


def kernel(query, key, value, sparse_indices, scale_value, block_table, actual_seq_lengths_query, actual_seq_lengths_kv, query_rope, key_rope, sparse_block_size, layout_query, layout_kv, sparse_mode, pre_tokens, next_tokens, attention_mode, return_softmax_lse):
    raise NotImplementedError("write your pallas kernel here")


# speedup vs baseline: 1.1246x; 1.1246x over previous
import functools

import jax
import jax.numpy as jnp
from jax.experimental import pallas as pl
from jax.experimental.pallas import tpu as pltpu


def _flash_body(si_ref, bt_ref, scale_ref, q_ref, qr_ref, k_ref, kr_ref,
                v_ref, o_ref, *, bs, ksel, nqb, nkb, heads):
    n = pl.program_id(0)
    qb = pl.program_id(1)
    batch = n // heads
    scale = scale_ref[0]
    q = q_ref[0]
    qr = qr_ref[0]
    base = (n * nqb + qb) * ksel
    offs = []
    parts = []
    for j in range(ksel):
        phys = bt_ref[batch * nkb + si_ref[base + j]]
        off = phys * bs
        offs.append(off)
        k = k_ref[0, pl.ds(off, bs), :]
        kr = kr_ref[0, pl.ds(off, bs), :]
        s = jnp.dot(q, k.T, preferred_element_type=jnp.float32)
        s += jnp.dot(qr, kr.T, preferred_element_type=jnp.float32)
        parts.append(s * scale)
    s_all = jnp.concatenate(parts, axis=1)
    m = jnp.max(s_all, axis=1, keepdims=True)
    p = jnp.exp(s_all - m)
    l = jnp.sum(p, axis=1, keepdims=True)
    acc = jnp.zeros(o_ref.shape[1:], jnp.float32)
    for j in range(ksel):
        v = v_ref[0, pl.ds(offs[j], bs), :]
        acc += jnp.dot(p[:, j * bs:(j + 1) * bs], v,
                       preferred_element_type=jnp.float32)
    o_ref[0] = acc / l


def kernel(query, key, value, sparse_indices, scale_value, block_table,
           actual_seq_lengths_query, actual_seq_lengths_kv, query_rope,
           key_rope, sparse_block_size, layout_query, layout_kv, sparse_mode,
           pre_tokens, next_tokens, attention_mode, return_softmax_lse):
    b, n, sq, d = query.shape
    dr = query_rope.shape[-1]
    skv = key.shape[2]
    nqb = sparse_indices.shape[2]
    ksel = sparse_indices.shape[3]
    bs = sq // nqb
    nkb = skv // bs
    bn = b * n

    q = query.reshape(bn, sq, d)
    qr = query_rope.reshape(bn, sq, dr)
    k = key.reshape(bn, skv, d)
    kr = key_rope.reshape(bn, skv, dr)
    v = value.reshape(bn, skv, d)
    si = sparse_indices.reshape(-1)
    bt = block_table.reshape(-1)
    scale = jnp.asarray(scale_value, jnp.float32).reshape(1)

    body = functools.partial(_flash_body, bs=bs, ksel=ksel, nqb=nqb,
                             nkb=nkb, heads=n)
    grid_spec = pltpu.PrefetchScalarGridSpec(
        num_scalar_prefetch=3,
        grid=(bn, nqb),
        in_specs=[
            pl.BlockSpec((1, bs, d), lambda h, qb, *_: (h, qb, 0)),
            pl.BlockSpec((1, bs, dr), lambda h, qb, *_: (h, qb, 0)),
            pl.BlockSpec((1, skv, d), lambda h, qb, *_: (h, 0, 0)),
            pl.BlockSpec((1, skv, dr), lambda h, qb, *_: (h, 0, 0)),
            pl.BlockSpec((1, skv, d), lambda h, qb, *_: (h, 0, 0)),
        ],
        out_specs=pl.BlockSpec((1, bs, d), lambda h, qb, *_: (h, qb, 0)),
    )
    out = pl.pallas_call(
        body,
        grid_spec=grid_spec,
        out_shape=jax.ShapeDtypeStruct((bn, sq, d), jnp.float32),
    )(si, bt, scale, q, qr, k, kr, v)
    return out.reshape(b, n, sq, d)


# speedup vs baseline: 2.0844x; 1.8535x over previous
import functools

import jax
import jax.numpy as jnp
from jax.experimental import pallas as pl
from jax.experimental.pallas import tpu as pltpu


def _flash_body(si_ref, bt_ref, scale_ref, q_ref, qr_ref, k_ref, kr_ref,
                v_ref, o_ref, *, bs, ksel, nqb, nkb, heads):
    h = pl.program_id(0)
    batch = h // heads
    scale = scale_ref[0]
    for qb in range(nqb):
        q = q_ref[0, qb * bs:(qb + 1) * bs, :].astype(jnp.bfloat16)
        qr = qr_ref[0, qb * bs:(qb + 1) * bs, :].astype(jnp.bfloat16)
        base = (h * nqb + qb) * ksel
        offs = []
        parts = []
        for j in range(ksel):
            phys = bt_ref[batch * nkb + si_ref[base + j]]
            off = phys * bs
            offs.append(off)
            k = k_ref[0, pl.ds(off, bs), :].astype(jnp.bfloat16)
            kr = kr_ref[0, pl.ds(off, bs), :].astype(jnp.bfloat16)
            s = jnp.dot(q, k.T, preferred_element_type=jnp.float32)
            s += jnp.dot(qr, kr.T, preferred_element_type=jnp.float32)
            parts.append(s * scale)
        s_all = jnp.concatenate(parts, axis=1)
        m = jnp.max(s_all, axis=1, keepdims=True)
        p = jnp.exp(s_all - m)
        l = jnp.sum(p, axis=1, keepdims=True)
        pb = p.astype(jnp.bfloat16)
        acc = jnp.zeros((bs, v_ref.shape[-1]), jnp.float32)
        for j in range(ksel):
            v = v_ref[0, pl.ds(offs[j], bs), :].astype(jnp.bfloat16)
            acc += jnp.dot(pb[:, j * bs:(j + 1) * bs], v,
                           preferred_element_type=jnp.float32)
        o_ref[0, qb * bs:(qb + 1) * bs, :] = acc / l


def kernel(query, key, value, sparse_indices, scale_value, block_table,
           actual_seq_lengths_query, actual_seq_lengths_kv, query_rope,
           key_rope, sparse_block_size, layout_query, layout_kv, sparse_mode,
           pre_tokens, next_tokens, attention_mode, return_softmax_lse):
    b, n, sq, d = query.shape
    dr = query_rope.shape[-1]
    skv = key.shape[2]
    nqb = sparse_indices.shape[2]
    ksel = sparse_indices.shape[3]
    bs = sq // nqb
    nkb = skv // bs
    bn = b * n

    q = query.reshape(bn, sq, d)
    qr = query_rope.reshape(bn, sq, dr)
    k = key.reshape(bn, skv, d)
    kr = key_rope.reshape(bn, skv, dr)
    v = value.reshape(bn, skv, d)
    si = sparse_indices.reshape(-1)
    bt = block_table.reshape(-1)
    scale = jnp.asarray(scale_value, jnp.float32).reshape(1)

    body = functools.partial(_flash_body, bs=bs, ksel=ksel, nqb=nqb,
                             nkb=nkb, heads=n)
    grid_spec = pltpu.PrefetchScalarGridSpec(
        num_scalar_prefetch=3,
        grid=(bn,),
        in_specs=[
            pl.BlockSpec((1, sq, d), lambda h, *_: (h, 0, 0)),
            pl.BlockSpec((1, sq, dr), lambda h, *_: (h, 0, 0)),
            pl.BlockSpec((1, skv, d), lambda h, *_: (h, 0, 0)),
            pl.BlockSpec((1, skv, dr), lambda h, *_: (h, 0, 0)),
            pl.BlockSpec((1, skv, d), lambda h, *_: (h, 0, 0)),
        ],
        out_specs=pl.BlockSpec((1, sq, d), lambda h, *_: (h, 0, 0)),
    )
    out = pl.pallas_call(
        body,
        grid_spec=grid_spec,
        out_shape=jax.ShapeDtypeStruct((bn, sq, d), jnp.float32),
    )(si, bt, scale, q, qr, k, kr, v)
    return out.reshape(b, n, sq, d)


# speedup vs baseline: 2.2896x; 1.0984x over previous
import functools

import jax
import jax.numpy as jnp
from jax.experimental import pallas as pl
from jax.experimental.pallas import tpu as pltpu


def _flash_body(si_ref, bt_ref, scale_ref, q_ref, qr_ref, k_ref, kr_ref,
                v_ref, o_ref, kf_s, vb_s, *, bs, ksel, nqb, nkb, heads):
    h = pl.program_id(0)
    batch = h // heads
    scale = scale_ref[0]
    d = v_ref.shape[-1]
    dr = kr_ref.shape[-1]
    kf_s[:, :d] = k_ref[0].astype(jnp.bfloat16)
    kf_s[:, d:] = kr_ref[0].astype(jnp.bfloat16)
    vb_s[...] = v_ref[0].astype(jnp.bfloat16)
    for qb in range(nqb):
        qf = jnp.concatenate(
            [q_ref[0, qb * bs:(qb + 1) * bs, :],
             qr_ref[0, qb * bs:(qb + 1) * bs, :]],
            axis=1).astype(jnp.bfloat16)
        base = (h * nqb + qb) * ksel
        offs = [bt_ref[batch * nkb + si_ref[base + j]] * bs
                for j in range(ksel)]
        kcat = jnp.concatenate([kf_s[pl.ds(off, bs), :] for off in offs],
                               axis=0)
        s = jnp.dot(qf, kcat.T, preferred_element_type=jnp.float32) * scale
        m = jnp.max(s, axis=1, keepdims=True)
        p = jnp.exp(s - m)
        l = jnp.sum(p, axis=1, keepdims=True)
        vcat = jnp.concatenate([vb_s[pl.ds(off, bs), :] for off in offs],
                               axis=0)
        acc = jnp.dot(p.astype(jnp.bfloat16), vcat,
                      preferred_element_type=jnp.float32)
        o_ref[0, qb * bs:(qb + 1) * bs, :] = acc / l


def kernel(query, key, value, sparse_indices, scale_value, block_table,
           actual_seq_lengths_query, actual_seq_lengths_kv, query_rope,
           key_rope, sparse_block_size, layout_query, layout_kv, sparse_mode,
           pre_tokens, next_tokens, attention_mode, return_softmax_lse):
    b, n, sq, d = query.shape
    dr = query_rope.shape[-1]
    skv = key.shape[2]
    nqb = sparse_indices.shape[2]
    ksel = sparse_indices.shape[3]
    bs = sq // nqb
    nkb = skv // bs
    bn = b * n

    q = query.reshape(bn, sq, d)
    qr = query_rope.reshape(bn, sq, dr)
    k = key.reshape(bn, skv, d)
    kr = key_rope.reshape(bn, skv, dr)
    v = value.reshape(bn, skv, d)
    si = sparse_indices.reshape(-1)
    bt = block_table.reshape(-1)
    scale = jnp.asarray(scale_value, jnp.float32).reshape(1)

    body = functools.partial(_flash_body, bs=bs, ksel=ksel, nqb=nqb,
                             nkb=nkb, heads=n)
    grid_spec = pltpu.PrefetchScalarGridSpec(
        num_scalar_prefetch=3,
        grid=(bn,),
        in_specs=[
            pl.BlockSpec((1, sq, d), lambda h, *_: (h, 0, 0)),
            pl.BlockSpec((1, sq, dr), lambda h, *_: (h, 0, 0)),
            pl.BlockSpec((1, skv, d), lambda h, *_: (h, 0, 0)),
            pl.BlockSpec((1, skv, dr), lambda h, *_: (h, 0, 0)),
            pl.BlockSpec((1, skv, d), lambda h, *_: (h, 0, 0)),
        ],
        out_specs=pl.BlockSpec((1, sq, d), lambda h, *_: (h, 0, 0)),
        scratch_shapes=[
            pltpu.VMEM((skv, d + dr), jnp.bfloat16),
            pltpu.VMEM((skv, d), jnp.bfloat16),
        ],
    )
    out = pl.pallas_call(
        body,
        grid_spec=grid_spec,
        out_shape=jax.ShapeDtypeStruct((bn, sq, d), jnp.float32),
    )(si, bt, scale, q, qr, k, kr, v)
    return out.reshape(b, n, sq, d)


# speedup vs baseline: 2.4439x; 1.0674x over previous
import functools

import jax
import jax.numpy as jnp
from jax.experimental import pallas as pl
from jax.experimental.pallas import tpu as pltpu


def _flash_body(si_ref, bt_ref, scale_ref, q_ref, qr_ref, k_ref, kr_ref,
                v_ref, o_ref, kf_s, vb_s, *, bs, ksel, nqb, nkb, heads):
    h = pl.program_id(0)
    batch = h // heads
    head = h % heads
    scale = scale_ref[0]
    d = v_ref.shape[-1]
    kf_s[:, :d] = k_ref[0, 0].astype(jnp.bfloat16)
    kf_s[:, d:] = kr_ref[0, 0].astype(jnp.bfloat16)
    vb_s[...] = v_ref[0, 0].astype(jnp.bfloat16)
    for qb in range(nqb):
        qf = jnp.concatenate(
            [q_ref[0, 0, qb * bs:(qb + 1) * bs, :],
             qr_ref[0, 0, qb * bs:(qb + 1) * bs, :]],
            axis=1).astype(jnp.bfloat16)
        offs = [bt_ref[batch, si_ref[batch, head, qb, j]] * bs
                for j in range(ksel)]
        kcat = jnp.concatenate([kf_s[pl.ds(off, bs), :] for off in offs],
                               axis=0)
        s = jnp.dot(qf, kcat.T, preferred_element_type=jnp.float32) * scale
        m = jnp.max(s, axis=1, keepdims=True)
        p = jnp.exp(s - m)
        l = jnp.sum(p, axis=1, keepdims=True)
        vcat = jnp.concatenate([vb_s[pl.ds(off, bs), :] for off in offs],
                               axis=0)
        acc = jnp.dot(p.astype(jnp.bfloat16), vcat,
                      preferred_element_type=jnp.float32)
        o_ref[0, 0, qb * bs:(qb + 1) * bs, :] = acc / l


def kernel(query, key, value, sparse_indices, scale_value, block_table,
           actual_seq_lengths_query, actual_seq_lengths_kv, query_rope,
           key_rope, sparse_block_size, layout_query, layout_kv, sparse_mode,
           pre_tokens, next_tokens, attention_mode, return_softmax_lse):
    b, n, sq, d = query.shape
    dr = query_rope.shape[-1]
    skv = key.shape[2]
    nqb = sparse_indices.shape[2]
    ksel = sparse_indices.shape[3]
    bs = sq // nqb
    nkb = skv // bs
    bn = b * n

    scale = jnp.asarray(scale_value, jnp.float32).reshape(1)

    body = functools.partial(_flash_body, bs=bs, ksel=ksel, nqb=nqb,
                             nkb=nkb, heads=n)

    def _hd(h):
        return (h // n, h % n)

    grid_spec = pltpu.PrefetchScalarGridSpec(
        num_scalar_prefetch=3,
        grid=(bn,),
        in_specs=[
            pl.BlockSpec((1, 1, sq, d), lambda h, *_: (*_hd(h), 0, 0)),
            pl.BlockSpec((1, 1, sq, dr), lambda h, *_: (*_hd(h), 0, 0)),
            pl.BlockSpec((1, 1, skv, d), lambda h, *_: (*_hd(h), 0, 0)),
            pl.BlockSpec((1, 1, skv, dr), lambda h, *_: (*_hd(h), 0, 0)),
            pl.BlockSpec((1, 1, skv, d), lambda h, *_: (*_hd(h), 0, 0)),
        ],
        out_specs=pl.BlockSpec((1, 1, sq, d), lambda h, *_: (*_hd(h), 0, 0)),
        scratch_shapes=[
            pltpu.VMEM((skv, d + dr), jnp.bfloat16),
            pltpu.VMEM((skv, d), jnp.bfloat16),
        ],
    )
    out = pl.pallas_call(
        body,
        grid_spec=grid_spec,
        out_shape=jax.ShapeDtypeStruct((b, n, sq, d), jnp.float32),
    )(sparse_indices, block_table, scale, query, query_rope, key, key_rope,
      value)
    return out


# speedup vs baseline: 3.1085x; 1.2720x over previous
import functools

import jax
import jax.numpy as jnp
from jax import lax
from jax.experimental import pallas as pl
from jax.experimental.pallas import tpu as pltpu


def _flash_body(si_ref, bt_ref, scale_ref, q_ref, qrt_ref, k_ref, krt_ref,
                v_ref, o_ref, kb_s, krt_s, vb_s, *, bs, ksel, nqb, nkb,
                heads):
    h = pl.program_id(0)
    batch = h // heads
    head = h % heads
    scale = scale_ref[0]
    kb_s[...] = k_ref[0, 0].astype(jnp.bfloat16)
    krt_s[...] = krt_ref[0, 0].astype(jnp.bfloat16)
    vb_s[...] = v_ref[0, 0].astype(jnp.bfloat16)
    for qb in range(nqb):
        q = q_ref[0, 0, qb * bs:(qb + 1) * bs, :].astype(jnp.bfloat16)
        qrt = qrt_ref[0, 0, :, qb * bs:(qb + 1) * bs].astype(jnp.bfloat16)
        offs = [bt_ref[batch, si_ref[batch, head, qb, j]] * bs
                for j in range(ksel)]
        kcat = jnp.concatenate([kb_s[pl.ds(off, bs), :] for off in offs],
                               axis=0)
        krtcat = jnp.concatenate([krt_s[:, pl.ds(off, bs)] for off in offs],
                                 axis=1)
        s = jnp.dot(q, kcat.T, preferred_element_type=jnp.float32)
        s += lax.dot_general(qrt, krtcat, (((0,), (0,)), ((), ())),
                             preferred_element_type=jnp.float32)
        s *= scale
        m = jnp.max(s, axis=1, keepdims=True)
        p = jnp.exp(s - m)
        l = jnp.sum(p, axis=1, keepdims=True)
        vcat = jnp.concatenate([vb_s[pl.ds(off, bs), :] for off in offs],
                               axis=0)
        acc = jnp.dot(p.astype(jnp.bfloat16), vcat,
                      preferred_element_type=jnp.float32)
        o_ref[0, 0, qb * bs:(qb + 1) * bs, :] = acc / l


def kernel(query, key, value, sparse_indices, scale_value, block_table,
           actual_seq_lengths_query, actual_seq_lengths_kv, query_rope,
           key_rope, sparse_block_size, layout_query, layout_kv, sparse_mode,
           pre_tokens, next_tokens, attention_mode, return_softmax_lse):
    b, n, sq, d = query.shape
    dr = query_rope.shape[-1]
    skv = key.shape[2]
    nqb = sparse_indices.shape[2]
    ksel = sparse_indices.shape[3]
    bs = sq // nqb
    nkb = skv // bs
    bn = b * n

    qrt = jnp.swapaxes(query_rope, 2, 3)
    krt = jnp.swapaxes(key_rope, 2, 3)
    scale = jnp.asarray(scale_value, jnp.float32).reshape(1)

    body = functools.partial(_flash_body, bs=bs, ksel=ksel, nqb=nqb,
                             nkb=nkb, heads=n)

    def _hd(h):
        return (h // n, h % n)

    grid_spec = pltpu.PrefetchScalarGridSpec(
        num_scalar_prefetch=3,
        grid=(bn,),
        in_specs=[
            pl.BlockSpec((1, 1, sq, d), lambda h, *_: (*_hd(h), 0, 0)),
            pl.BlockSpec((1, 1, dr, sq), lambda h, *_: (*_hd(h), 0, 0)),
            pl.BlockSpec((1, 1, skv, d), lambda h, *_: (*_hd(h), 0, 0)),
            pl.BlockSpec((1, 1, dr, skv), lambda h, *_: (*_hd(h), 0, 0)),
            pl.BlockSpec((1, 1, skv, d), lambda h, *_: (*_hd(h), 0, 0)),
        ],
        out_specs=pl.BlockSpec((1, 1, sq, d), lambda h, *_: (*_hd(h), 0, 0)),
        scratch_shapes=[
            pltpu.VMEM((skv, d), jnp.bfloat16),
            pltpu.VMEM((dr, skv), jnp.bfloat16),
            pltpu.VMEM((skv, d), jnp.bfloat16),
        ],
    )
    out = pl.pallas_call(
        body,
        grid_spec=grid_spec,
        out_shape=jax.ShapeDtypeStruct((b, n, sq, d), jnp.float32),
    )(sparse_indices, block_table, scale, query, qrt, key, krt, value)
    return out


# speedup vs baseline: 3.1294x; 1.0067x over previous
import functools

import jax
import jax.numpy as jnp
from jax import lax
from jax.experimental import pallas as pl
from jax.experimental.pallas import tpu as pltpu


def _flash_body(si_ref, bt_ref, scale_ref, q_ref, qrt_ref, k_ref, krt_ref,
                v_ref, o_ref, kb_s, krt_s, vb_s, *, bs, ksel, nqb, nkb,
                heads):
    h = pl.program_id(0)
    batch = h // heads
    head = h % heads
    scale2 = scale_ref[0] * 1.4426950408889634
    kb_s[...] = k_ref[0, 0].astype(jnp.bfloat16)
    krt_s[...] = krt_ref[0, 0].astype(jnp.bfloat16)
    vb_s[...] = v_ref[0, 0].astype(jnp.bfloat16)
    for qb in range(nqb):
        q = (q_ref[0, 0, qb * bs:(qb + 1) * bs, :] *
             scale2).astype(jnp.bfloat16)
        qrt = (qrt_ref[0, 0, :, qb * bs:(qb + 1) * bs] *
               scale2).astype(jnp.bfloat16)
        offs = [bt_ref[batch, si_ref[batch, head, qb, j]] * bs
                for j in range(ksel)]
        kcat = jnp.concatenate([kb_s[pl.ds(off, bs), :] for off in offs],
                               axis=0)
        krtcat = jnp.concatenate([krt_s[:, pl.ds(off, bs)] for off in offs],
                                 axis=1)
        s = jnp.dot(q, kcat.T, preferred_element_type=jnp.float32)
        s += lax.dot_general(qrt, krtcat, (((0,), (0,)), ((), ())),
                             preferred_element_type=jnp.float32)
        m = jnp.max(s, axis=1, keepdims=True)
        p = jnp.exp2(s - m)
        l = jnp.sum(p, axis=1, keepdims=True)
        vcat = jnp.concatenate([vb_s[pl.ds(off, bs), :] for off in offs],
                               axis=0)
        acc = jnp.dot(p.astype(jnp.bfloat16), vcat,
                      preferred_element_type=jnp.float32)
        o_ref[0, 0, qb * bs:(qb + 1) * bs, :] = acc / l


def kernel(query, key, value, sparse_indices, scale_value, block_table,
           actual_seq_lengths_query, actual_seq_lengths_kv, query_rope,
           key_rope, sparse_block_size, layout_query, layout_kv, sparse_mode,
           pre_tokens, next_tokens, attention_mode, return_softmax_lse):
    b, n, sq, d = query.shape
    dr = query_rope.shape[-1]
    skv = key.shape[2]
    nqb = sparse_indices.shape[2]
    ksel = sparse_indices.shape[3]
    bs = sq // nqb
    nkb = skv // bs
    bn = b * n

    qrt = jnp.swapaxes(query_rope, 2, 3)
    krt = jnp.swapaxes(key_rope, 2, 3)
    scale = jnp.asarray(scale_value, jnp.float32).reshape(1)

    body = functools.partial(_flash_body, bs=bs, ksel=ksel, nqb=nqb,
                             nkb=nkb, heads=n)

    def _hd(h):
        return (h // n, h % n)

    grid_spec = pltpu.PrefetchScalarGridSpec(
        num_scalar_prefetch=3,
        grid=(bn,),
        in_specs=[
            pl.BlockSpec((1, 1, sq, d), lambda h, *_: (*_hd(h), 0, 0)),
            pl.BlockSpec((1, 1, dr, sq), lambda h, *_: (*_hd(h), 0, 0)),
            pl.BlockSpec((1, 1, skv, d), lambda h, *_: (*_hd(h), 0, 0)),
            pl.BlockSpec((1, 1, dr, skv), lambda h, *_: (*_hd(h), 0, 0)),
            pl.BlockSpec((1, 1, skv, d), lambda h, *_: (*_hd(h), 0, 0)),
        ],
        out_specs=pl.BlockSpec((1, 1, sq, d), lambda h, *_: (*_hd(h), 0, 0)),
        scratch_shapes=[
            pltpu.VMEM((skv, d), jnp.bfloat16),
            pltpu.VMEM((dr, skv), jnp.bfloat16),
            pltpu.VMEM((skv, d), jnp.bfloat16),
        ],
    )
    out = pl.pallas_call(
        body,
        grid_spec=grid_spec,
        out_shape=jax.ShapeDtypeStruct((b, n, sq, d), jnp.float32),
    )(sparse_indices, block_table, scale, query, qrt, key, krt, value)
    return out
